# Initial kernel scaffold; baseline (speedup 1.0000x reference)
import jax
import jax.numpy as jnp
from jax.experimental import pallas as pl


def kernel(pos, edge_index, W1, b1, p1, W2, b2, W3, b3, p2, Wfc, bfc):
    raise NotImplementedError("write your pallas kernel here")


# speedup vs baseline: 1.9679x; 1.9679x over previous
import functools

import jax
import jax.numpy as jnp
from jax import lax
from jax.experimental import pallas as pl
from jax.experimental.pallas import tpu as pltpu


def _leaky(x):
    return jnp.where(x >= 0, x, 0.01 * x)


def _fc_body(flat_ref, w_ref, b_ref, o_ref):
    o_ref[...] = jnp.dot(flat_ref[...], w_ref[...],
                         preferred_element_type=jnp.float32) + b_ref[...]


def _fc(flat, Wfc, bfc):
    out = pl.pallas_call(
        _fc_body,
        out_shape=jax.ShapeDtypeStruct((1, 128), jnp.float32),
    )(flat.reshape(1, -1), Wfc, bfc.reshape(1, -1))
    return out.reshape(-1)


def kernel(pos, edge_index, W1, b1, p1, W2, b2, W3, b3, p2, Wfc, bfc):
    src = edge_index[0]
    dst = edge_index[1]
    N = pos.shape[0]
    E = src.shape[0]

    lin1 = pos @ W1
    deg1 = 1.0 + jax.ops.segment_sum(jnp.ones((E,), jnp.float32), dst,
                                     num_segments=N)
    dinv1 = lax.rsqrt(deg1)
    lins1 = lin1 * dinv1[:, None]
    acc1 = jax.ops.segment_sum(lins1[src], dst, num_segments=N)
    x1 = _leaky(dinv1[:, None] * acc1 + lin1 * (dinv1 * dinv1)[:, None] + b1)

    score1 = (x1 @ p1) / jnp.linalg.norm(p1)
    _, perm1 = lax.top_k(score1, 4096)
    gate1 = jnp.tanh(score1[perm1])
    xs = x1[perm1] * gate1[:, None]
    mapping = jnp.full((N,), -1, jnp.int32).at[perm1].set(
        jnp.arange(4096, dtype=jnp.int32))
    ns = mapping[src]
    nd = mapping[dst]
    valid = (ns >= 0) & (nd >= 0)
    ns = jnp.where(valid, ns, 4096)
    nd = jnp.where(valid, nd, 4096)

    deg2 = 1.0 + jax.ops.segment_sum(valid.astype(jnp.float32), nd,
                                     num_segments=4097)[:4096]
    dinv2 = lax.rsqrt(deg2)
    lin2 = xs @ W2
    lins2 = jnp.concatenate([lin2 * dinv2[:, None],
                             jnp.zeros((1, 32), jnp.float32)], axis=0)
    acc2 = jax.ops.segment_sum(lins2[ns], nd, num_segments=4097)[:4096]
    x2 = _leaky(dinv2[:, None] * acc2 + lin2 * (dinv2 * dinv2)[:, None] + b2)

    lin3 = x2 @ W3
    lins3 = jnp.concatenate([lin3 * dinv2[:, None],
                             jnp.zeros((1, 32), jnp.float32)], axis=0)
    acc3 = jax.ops.segment_sum(lins3[ns], nd, num_segments=4097)[:4096]
    x3 = _leaky(dinv2[:, None] * acc3 + lin3 * (dinv2 * dinv2)[:, None] + b3)

    score2 = (x3 @ p2) / jnp.linalg.norm(p2)
    _, perm2 = lax.top_k(score2, 128)
    xf = x3[perm2] * jnp.tanh(score2[perm2])[:, None]
    flat = xf.T.reshape(-1)
    return _fc(flat, Wfc, bfc)


# speedup vs baseline: 2.3980x; 1.2186x over previous
import functools

import jax
import jax.numpy as jnp
from jax import lax
from jax.experimental import pallas as pl
from jax.experimental.pallas import tpu as pltpu
from jax.experimental.pallas import tpu_sc as plsc

_NTILES = 16
_NCORES = 2
_LANE = 16


def _mesh():
    return plsc.VectorSubcoreMesh(core_axis_name="c", subcore_axis_name="s",
                                  num_cores=_NCORES, num_subcores=_NTILES)


def _fill_f32(buf, n, value):
    v = jnp.full((_LANE,), value, jnp.float32)

    def body(i, carry):
        buf[pl.ds(i * _LANE, _LANE)] = v
        return carry

    lax.fori_loop(0, n // _LANE, body, 0)


def _zero_rows(buf, nrows):
    z = jnp.zeros((_LANE,), jnp.float32)

    def body(i, carry):
        buf[i, :] = z
        return carry

    lax.fori_loop(0, nrows, body, 0)


@functools.lru_cache(maxsize=None)
def _make_deg_hist(n_pad, nchunks, interpret=False):
    stripe = n_pad // _NTILES

    @functools.partial(
        pl.kernel,
        out_type=jax.ShapeDtypeStruct((_NCORES * n_pad,), jnp.float32),
        mesh=_mesh(),
        scratch_types=[
            pltpu.VMEM((nchunks, 128), jnp.int32),
            pltpu.VMEM((128,), jnp.float32),
            pltpu.VMEM((stripe,), jnp.float32),
            pltpu.VMEM_SHARED((n_pad,), jnp.float32),
        ],
        compiler_params=pltpu.CompilerParams(use_tc_tiling_on_sc=False),
        interpret=interpret,
    )
    def deg_hist(dst_hbm, out_hbm, idx_v, ones_v, zb_v, hist_sh):
        c = lax.axis_index("c")
        s = lax.axis_index("s")
        wid = c * _NTILES + s
        _fill_f32(ones_v, 128, 1.0)
        _fill_f32(zb_v, stripe, 0.0)
        pltpu.sync_copy(dst_hbm.at[pl.ds(wid * nchunks, nchunks)], idx_v)
        pltpu.sync_copy(zb_v, hist_sh.at[pl.ds(s * stripe, stripe)])
        plsc.subcore_barrier()

        def body(j, carry):
            pltpu.sync_copy(ones_v, hist_sh.at[idx_v.at[j]], add=True)
            return carry

        lax.fori_loop(0, nchunks, body, 0)
        plsc.subcore_barrier()
        pltpu.sync_copy(hist_sh.at[pl.ds(s * stripe, stripe)],
                        out_hbm.at[pl.ds(c * n_pad + s * stripe, stripe)])

    return deg_hist


@functools.lru_cache(maxsize=None)
def _make_edge_acc(n_pad, width, nchunks, interpret=False):
    stripe = n_pad // _NTILES
    zrows = 400 if stripe % 400 == 0 else stripe
    nz = stripe // zrows

    @functools.partial(
        pl.kernel,
        out_type=[jax.ShapeDtypeStruct((n_pad, width), jnp.float32),
                  jax.ShapeDtypeStruct((n_pad, width), jnp.float32)],
        mesh=_mesh(),
        scratch_types=[
            pltpu.VMEM((nchunks, 128), jnp.int32),
            pltpu.VMEM((nchunks, 128), jnp.int32),
            pltpu.VMEM((128, width), jnp.float32),
            pltpu.VMEM((zrows, width), jnp.float32),
            pltpu.SemaphoreType.DMA,
            pltpu.VMEM_SHARED((n_pad, width), jnp.float32),
        ],
        compiler_params=pltpu.CompilerParams(use_tc_tiling_on_sc=False),
        interpret=interpret,
    )
    def edge_acc(rows_hbm, src_hbm, dst_hbm, out0_hbm, out1_hbm,
                 sidx_v, didx_v, rowbuf, zb_v, sem, acc_sh):
        c = lax.axis_index("c")
        s = lax.axis_index("s")
        wid = c * _NTILES + s
        _zero_rows(zb_v, zrows)
        pltpu.sync_copy(src_hbm.at[pl.ds(wid * nchunks, nchunks)], sidx_v)
        pltpu.sync_copy(dst_hbm.at[pl.ds(wid * nchunks, nchunks)], didx_v)

        def zbody(k, carry):
            pltpu.sync_copy(
                zb_v, acc_sh.at[pl.ds(s * stripe + k * zrows, zrows)])
            return carry

        lax.fori_loop(0, nz, zbody, 0)
        plsc.subcore_barrier()

        def body(j, carry):
            pltpu.async_copy(rows_hbm.at[sidx_v.at[j]], rowbuf, sem).wait()
            pltpu.sync_copy(rowbuf, acc_sh.at[didx_v.at[j]], add=True)
            return carry

        lax.fori_loop(0, nchunks, body, 0)
        plsc.subcore_barrier()

        @pl.when(c == 0)
        def _():
            pltpu.sync_copy(acc_sh.at[pl.ds(s * stripe, stripe)],
                            out0_hbm.at[pl.ds(s * stripe, stripe)])

        @pl.when(c == 1)
        def _():
            pltpu.sync_copy(acc_sh.at[pl.ds(s * stripe, stripe)],
                            out1_hbm.at[pl.ds(s * stripe, stripe)])

    return edge_acc


def _leaky(x):
    return jnp.where(x >= 0, x, 0.01 * x)


def _fc_body(flat_ref, w_ref, b_ref, o_ref):
    o_ref[...] = jnp.dot(flat_ref[...], w_ref[...],
                         preferred_element_type=jnp.float32) + b_ref[...]


def _fc(flat, Wfc, bfc):
    out = pl.pallas_call(
        _fc_body,
        out_shape=jax.ShapeDtypeStruct((1, 128), jnp.float32),
    )(flat.reshape(1, -1), Wfc, bfc.reshape(1, -1))
    return out.reshape(-1)


def kernel(pos, edge_index, W1, b1, p1, W2, b2, W3, b3, p2, Wfc, bfc):
    src = edge_index[0]
    dst = edge_index[1]
    N = pos.shape[0]
    E = src.shape[0]

    n_pad = 51200
    ept = 25600
    e_pad = 32 * ept
    nchunks = ept // 128
    dummy = jnp.int32(N)

    src_p = jnp.full((e_pad,), dummy, jnp.int32).at[:E].set(src)
    dst_p = jnp.full((e_pad,), dummy, jnp.int32).at[:E].set(dst)
    src2d = src_p.reshape(e_pad // 128, 128)
    dst2d = dst_p.reshape(e_pad // 128, 128)

    lin1 = pos @ W1
    hist = _make_deg_hist(n_pad, nchunks)(dst2d).reshape(2, n_pad)
    deg1 = 1.0 + (hist[0] + hist[1])[:N]
    dinv1 = lax.rsqrt(deg1)
    lins1 = jnp.zeros((n_pad, 16), jnp.float32).at[:N].set(
        lin1 * dinv1[:, None])
    acc_a, acc_b = _make_edge_acc(n_pad, 16, nchunks)(lins1, src2d, dst2d)
    acc1 = (acc_a + acc_b)[:N]
    x1 = _leaky(dinv1[:, None] * acc1 + lin1 * (dinv1 * dinv1)[:, None] + b1)

    score1 = (x1 @ p1) / jnp.linalg.norm(p1)
    _, perm1 = lax.top_k(score1, 4096)
    gate1 = jnp.tanh(score1[perm1])
    xs = x1[perm1] * gate1[:, None]
    mapping = jnp.full((N,), -1, jnp.int32).at[perm1].set(
        jnp.arange(4096, dtype=jnp.int32))
    ns = mapping[src]
    nd = mapping[dst]
    valid = (ns >= 0) & (nd >= 0)
    ns = jnp.where(valid, ns, 4096)
    nd = jnp.where(valid, nd, 4096)

    deg2 = 1.0 + jax.ops.segment_sum(valid.astype(jnp.float32), nd,
                                     num_segments=4097)[:4096]
    dinv2 = lax.rsqrt(deg2)
    lin2 = xs @ W2
    lins2 = jnp.concatenate([lin2 * dinv2[:, None],
                             jnp.zeros((1, 32), jnp.float32)], axis=0)
    acc2 = jax.ops.segment_sum(lins2[ns], nd, num_segments=4097)[:4096]
    x2 = _leaky(dinv2[:, None] * acc2 + lin2 * (dinv2 * dinv2)[:, None] + b2)

    lin3 = x2 @ W3
    lins3 = jnp.concatenate([lin3 * dinv2[:, None],
                             jnp.zeros((1, 32), jnp.float32)], axis=0)
    acc3 = jax.ops.segment_sum(lins3[ns], nd, num_segments=4097)[:4096]
    x3 = _leaky(dinv2[:, None] * acc3 + lin3 * (dinv2 * dinv2)[:, None] + b3)

    score2 = (x3 @ p2) / jnp.linalg.norm(p2)
    _, perm2 = lax.top_k(score2, 128)
    xf = x3[perm2] * jnp.tanh(score2[perm2])[:, None]
    flat = xf.T.reshape(-1)
    return _fc(flat, Wfc, bfc)


# speedup vs baseline: 46.7147x; 19.4805x over previous
import functools

import jax
import jax.numpy as jnp
from jax import lax
from jax.experimental import pallas as pl
from jax.experimental.pallas import tpu as pltpu
from jax.experimental.pallas import tpu_sc as plsc

_NTILES = 16
_NCORES = 2
_LANE = 16


def _mesh():
    return plsc.VectorSubcoreMesh(core_axis_name="c", subcore_axis_name="s",
                                  num_cores=_NCORES, num_subcores=_NTILES)


def _fill_f32(buf, n, value):
    v = jnp.full((_LANE,), value, jnp.float32)

    def body(i, carry):
        buf[pl.ds(i * _LANE, _LANE)] = v
        return carry

    lax.fori_loop(0, n // _LANE, body, 0)


def _zero_rows(buf, nrows, width):
    z = jnp.zeros((_LANE,), jnp.float32)

    def body(i, carry):
        for k in range(width // _LANE):
            buf[i, pl.ds(k * _LANE, _LANE)] = z
        return carry

    lax.fori_loop(0, nrows, body, 0)


@functools.lru_cache(maxsize=None)
def _make_deg_hist(n_pad, nchunks, interpret=False):
    stripe = n_pad // _NTILES

    @functools.partial(
        pl.kernel,
        out_type=jax.ShapeDtypeStruct((_NCORES * n_pad,), jnp.float32),
        mesh=_mesh(),
        scratch_types=[
            pltpu.VMEM((nchunks, 128), jnp.int32),
            pltpu.VMEM((128,), jnp.float32),
            pltpu.VMEM((stripe,), jnp.float32),
            pltpu.VMEM_SHARED((n_pad,), jnp.float32),
        ],
        compiler_params=pltpu.CompilerParams(use_tc_tiling_on_sc=False,
                                             needs_layout_passes=False),
        interpret=interpret,
    )
    def deg_hist(dst_hbm, out_hbm, idx_v, ones_v, zb_v, hist_sh):
        c = lax.axis_index("c")
        s = lax.axis_index("s")
        wid = c * _NTILES + s
        _fill_f32(ones_v, 128, 1.0)
        _fill_f32(zb_v, stripe, 0.0)
        pltpu.sync_copy(dst_hbm.at[pl.ds(wid * nchunks, nchunks)], idx_v)
        pltpu.sync_copy(zb_v, hist_sh.at[pl.ds(s * stripe, stripe)])
        plsc.subcore_barrier()

        def body(j, carry):
            pltpu.sync_copy(ones_v, hist_sh.at[idx_v.at[j]], add=True)
            return carry

        lax.fori_loop(0, nchunks, body, 0)
        plsc.subcore_barrier()
        pltpu.sync_copy(hist_sh.at[pl.ds(s * stripe, stripe)],
                        out_hbm.at[pl.ds(c * n_pad + s * stripe, stripe)])

    return deg_hist


@functools.lru_cache(maxsize=None)
def _make_edge_acc(n_pad, width, nchunks, interpret=False):
    stripe = n_pad // _NTILES
    zrows = 400 if stripe % 400 == 0 else stripe
    nz = stripe // zrows

    @functools.partial(
        pl.kernel,
        out_type=[jax.ShapeDtypeStruct((n_pad, width), jnp.float32),
                  jax.ShapeDtypeStruct((n_pad, width), jnp.float32)],
        mesh=_mesh(),
        scratch_types=[
            pltpu.VMEM((nchunks, 128), jnp.int32),
            pltpu.VMEM((nchunks, 128), jnp.int32),
            pltpu.VMEM((128, width), jnp.float32),
            pltpu.VMEM((zrows, width), jnp.float32),
            pltpu.SemaphoreType.DMA,
            pltpu.VMEM_SHARED((n_pad, width), jnp.float32),
        ],
        compiler_params=pltpu.CompilerParams(use_tc_tiling_on_sc=False,
                                             needs_layout_passes=False),
        interpret=interpret,
    )
    def edge_acc(rows_hbm, src_hbm, dst_hbm, out0_hbm, out1_hbm,
                 sidx_v, didx_v, rowbuf, zb_v, sem, acc_sh):
        c = lax.axis_index("c")
        s = lax.axis_index("s")
        wid = c * _NTILES + s
        _zero_rows(zb_v, zrows, width)
        pltpu.sync_copy(src_hbm.at[pl.ds(wid * nchunks, nchunks)], sidx_v)
        pltpu.sync_copy(dst_hbm.at[pl.ds(wid * nchunks, nchunks)], didx_v)

        def zbody(k, carry):
            pltpu.sync_copy(
                zb_v, acc_sh.at[pl.ds(s * stripe + k * zrows, zrows)])
            return carry

        lax.fori_loop(0, nz, zbody, 0)
        plsc.subcore_barrier()

        def body(j, carry):
            pltpu.async_copy(rows_hbm.at[sidx_v.at[j]], rowbuf, sem).wait()
            pltpu.sync_copy(rowbuf, acc_sh.at[didx_v.at[j]], add=True)
            return carry

        lax.fori_loop(0, nchunks, body, 0)
        plsc.subcore_barrier()

        @pl.when(c == 0)
        def _():
            pltpu.sync_copy(acc_sh.at[pl.ds(s * stripe, stripe)],
                            out0_hbm.at[pl.ds(s * stripe, stripe)])

        @pl.when(c == 1)
        def _():
            pltpu.sync_copy(acc_sh.at[pl.ds(s * stripe, stripe)],
                            out1_hbm.at[pl.ds(s * stripe, stripe)])

    return edge_acc


@functools.lru_cache(maxsize=None)
def _make_remap_compact(n_pad, nchunks, cap_chunks, nbins, dummy_id,
                        interpret=False):
    stripe = n_pad // _NTILES
    cap = cap_chunks * 128
    nw = _NCORES * _NTILES

    @functools.partial(
        pl.kernel,
        out_type=[jax.ShapeDtypeStruct((nw * cap,), jnp.int32),
                  jax.ShapeDtypeStruct((nw * cap,), jnp.int32),
                  jax.ShapeDtypeStruct((nw * 16,), jnp.int32),
                  jax.ShapeDtypeStruct((nw * nbins,), jnp.float32)],
        mesh=_mesh(),
        scratch_types=[
            pltpu.VMEM((nchunks, 128), jnp.int32),
            pltpu.VMEM((nchunks, 128), jnp.int32),
            pltpu.VMEM((128,), jnp.int32),
            pltpu.VMEM((128,), jnp.int32),
            pltpu.VMEM((cap + 128,), jnp.int32),
            pltpu.VMEM((cap + 128,), jnp.int32),
            pltpu.VMEM((nbins,), jnp.float32),
            pltpu.VMEM((16,), jnp.int32),
            pltpu.SemaphoreType.DMA,
        ],
        compiler_params=pltpu.CompilerParams(use_tc_tiling_on_sc=False,
                                             needs_layout_passes=False),
        interpret=interpret,
    )
    def remap_compact(map_hbm, src_hbm, dst_hbm,
                      ns_hbm, nd_hbm, cnt_hbm, hist_hbm,
                      sidx_v, didx_v, ms_v, md_v, nsb_v, ndb_v, hist_v,
                      cnt_v, sem):
        c = lax.axis_index("c")
        s = lax.axis_index("s")
        wid = c * _NTILES + s
        dummy = jnp.full((_LANE,), dummy_id, jnp.int32)
        ones = jnp.ones((_LANE,), jnp.float32)
        _fill_f32(hist_v, nbins, 0.0)
        pltpu.sync_copy(src_hbm.at[pl.ds(wid * nchunks, nchunks)], sidx_v)
        pltpu.sync_copy(dst_hbm.at[pl.ds(wid * nchunks, nchunks)], didx_v)
        def chunk(j, pos):
            pltpu.async_copy(map_hbm.at[sidx_v.at[j]], ms_v, sem).wait()
            pltpu.async_copy(map_hbm.at[didx_v.at[j]], md_v, sem).wait()
            for v in range(8):
                ms = ms_v[pl.ds(v * _LANE, _LANE)]
                md = md_v[pl.ds(v * _LANE, _LANE)]
                m = (ms >= 0) & (md >= 0)
                mi = m.astype(jnp.int32)
                slot = pos + plsc.cumsum(mi) - mi
                plsc.store_scatter(nsb_v, [slot], ms, mask=m)
                plsc.store_scatter(ndb_v, [slot], md, mask=m)
                plsc.addupdate_scatter(hist_v, [md], ones, mask=m)
                pos = pos + jnp.sum(mi)
            return pos

        pos = lax.fori_loop(0, nchunks, chunk, jnp.int32(0))
        for v in range(8):
            nsb_v[pl.ds(pos + v * _LANE, _LANE)] = dummy
            ndb_v[pl.ds(pos + v * _LANE, _LANE)] = dummy
        cnt_v[...] = jnp.full((_LANE,), (pos + 127) // 128, jnp.int32)
        pltpu.sync_copy(nsb_v.at[pl.ds(0, cap)], ns_hbm.at[pl.ds(wid * cap, cap)])
        pltpu.sync_copy(ndb_v.at[pl.ds(0, cap)], nd_hbm.at[pl.ds(wid * cap, cap)])
        pltpu.sync_copy(cnt_v, cnt_hbm.at[pl.ds(wid * 16, 16)])
        pltpu.sync_copy(hist_v, hist_hbm.at[pl.ds(wid * nbins, nbins)])

    return remap_compact


@functools.lru_cache(maxsize=None)
def _make_edge_acc_dyn(n_pad, width, cap_chunks, interpret=False):
    stripe = n_pad // _NTILES
    nw = _NCORES * _NTILES

    @functools.partial(
        pl.kernel,
        out_type=[jax.ShapeDtypeStruct((n_pad, width), jnp.float32),
                  jax.ShapeDtypeStruct((n_pad, width), jnp.float32)],
        mesh=_mesh(),
        scratch_types=[
            pltpu.VMEM((cap_chunks, 128), jnp.int32),
            pltpu.VMEM((cap_chunks, 128), jnp.int32),
            pltpu.VMEM((128, width), jnp.float32),
            pltpu.VMEM((stripe, width), jnp.float32),
            pltpu.VMEM((16,), jnp.int32),
            pltpu.SemaphoreType.DMA,
            pltpu.VMEM_SHARED((n_pad, width), jnp.float32),
        ],
        compiler_params=pltpu.CompilerParams(use_tc_tiling_on_sc=False,
                                             needs_layout_passes=False),
        interpret=interpret,
    )
    def edge_acc_dyn(rows_hbm, src_hbm, dst_hbm, cnt_hbm, out0_hbm, out1_hbm,
                     sidx_v, didx_v, rowbuf, zb_v, cnt_v, sem, acc_sh):
        c = lax.axis_index("c")
        s = lax.axis_index("s")
        wid = c * _NTILES + s
        _zero_rows(zb_v, stripe, width)
        pltpu.sync_copy(src_hbm.at[pl.ds(wid * cap_chunks, cap_chunks)], sidx_v)
        pltpu.sync_copy(dst_hbm.at[pl.ds(wid * cap_chunks, cap_chunks)], didx_v)
        pltpu.sync_copy(cnt_hbm.at[pl.ds(wid * 16, 16)], cnt_v)
        pltpu.sync_copy(zb_v, acc_sh.at[pl.ds(s * stripe, stripe)])
        plsc.subcore_barrier()
        ncj = jnp.max(cnt_v[...])

        def body(j, carry):
            pltpu.async_copy(rows_hbm.at[sidx_v.at[j]], rowbuf, sem).wait()
            pltpu.sync_copy(rowbuf, acc_sh.at[didx_v.at[j]], add=True)
            return carry

        lax.fori_loop(0, ncj, body, 0)
        plsc.subcore_barrier()

        @pl.when(c == 0)
        def _():
            pltpu.sync_copy(acc_sh.at[pl.ds(s * stripe, stripe)],
                            out0_hbm.at[pl.ds(s * stripe, stripe)])

        @pl.when(c == 1)
        def _():
            pltpu.sync_copy(acc_sh.at[pl.ds(s * stripe, stripe)],
                            out1_hbm.at[pl.ds(s * stripe, stripe)])

    return edge_acc_dyn


def _leaky(x):
    return jnp.where(x >= 0, x, 0.01 * x)


def _fc_body(flat_ref, w_ref, b_ref, o_ref):
    o_ref[...] = jnp.dot(flat_ref[...], w_ref[...],
                         preferred_element_type=jnp.float32) + b_ref[...]


def _fc(flat, Wfc, bfc):
    out = pl.pallas_call(
        _fc_body,
        out_shape=jax.ShapeDtypeStruct((1, 128), jnp.float32),
    )(flat.reshape(1, -1), Wfc, bfc.reshape(1, -1))
    return out.reshape(-1)


def kernel(pos, edge_index, W1, b1, p1, W2, b2, W3, b3, p2, Wfc, bfc):
    src = edge_index[0]
    dst = edge_index[1]
    N = pos.shape[0]
    E = src.shape[0]

    n_pad = 51200
    ept = 25600
    e_pad = 32 * ept
    nchunks = ept // 128
    dummy = jnp.int32(N)

    src_p = jnp.full((e_pad,), dummy, jnp.int32).at[:E].set(src)
    dst_p = jnp.full((e_pad,), dummy, jnp.int32).at[:E].set(dst)
    src2d = src_p.reshape(e_pad // 128, 128)
    dst2d = dst_p.reshape(e_pad // 128, 128)

    lin1 = pos @ W1
    hist = _make_deg_hist(n_pad, nchunks)(dst2d).reshape(2, n_pad)
    deg1 = 1.0 + (hist[0] + hist[1])[:N]
    dinv1 = lax.rsqrt(deg1)
    lins1 = jnp.zeros((n_pad, 16), jnp.float32).at[:N].set(
        lin1 * dinv1[:, None])
    acc_a, acc_b = _make_edge_acc(n_pad, 16, nchunks)(lins1, src2d, dst2d)
    acc1 = (acc_a + acc_b)[:N]
    x1 = _leaky(dinv1[:, None] * acc1 + lin1 * (dinv1 * dinv1)[:, None] + b1)

    score1 = (x1 @ p1) / jnp.linalg.norm(p1)
    _, perm1 = lax.top_k(score1, 4096)
    gate1 = jnp.tanh(score1[perm1])
    xs = x1[perm1] * gate1[:, None]
    mapping = jnp.full((n_pad,), -1, jnp.int32).at[perm1].set(
        jnp.arange(4096, dtype=jnp.int32))

    cap_chunks = nchunks + 2
    nbins = 8192
    n2pad = 4224
    ns_c, nd_c, cnts, histp = _make_remap_compact(
        n_pad, nchunks, cap_chunks, nbins, 4096)(mapping, src2d, dst2d)
    ns2d = ns_c.reshape(-1, 128)
    nd2d = nd_c.reshape(-1, 128)
    deg2 = 1.0 + jnp.sum(histp.reshape(32, nbins), axis=0)[:4096]
    dinv2 = lax.rsqrt(deg2)

    lin2 = xs @ W2
    lins2 = jnp.zeros((n2pad, 32), jnp.float32).at[:4096].set(
        lin2 * dinv2[:, None])
    acc2a, acc2b = _make_edge_acc_dyn(n2pad, 32, cap_chunks)(
        lins2, ns2d, nd2d, cnts)
    acc2 = (acc2a + acc2b)[:4096]
    x2 = _leaky(dinv2[:, None] * acc2 + lin2 * (dinv2 * dinv2)[:, None] + b2)

    lin3 = x2 @ W3
    lins3 = jnp.zeros((n2pad, 32), jnp.float32).at[:4096].set(
        lin3 * dinv2[:, None])
    acc3a, acc3b = _make_edge_acc_dyn(n2pad, 32, cap_chunks)(
        lins3, ns2d, nd2d, cnts)
    acc3 = (acc3a + acc3b)[:4096]
    x3 = _leaky(dinv2[:, None] * acc3 + lin3 * (dinv2 * dinv2)[:, None] + b3)

    score2 = (x3 @ p2) / jnp.linalg.norm(p2)
    _, perm2 = lax.top_k(score2, 128)
    xf = x3[perm2] * jnp.tanh(score2[perm2])[:, None]
    flat = xf.T.reshape(-1)
    return _fc(flat, Wfc, bfc)


# speedup vs baseline: 68.9450x; 1.4759x over previous
import functools

import jax
import jax.numpy as jnp
from jax import lax
from jax.experimental import pallas as pl
from jax.experimental.pallas import tpu as pltpu
from jax.experimental.pallas import tpu_sc as plsc

_NTILES = 16
_NCORES = 2
_LANE = 16


def _mesh():
    return plsc.VectorSubcoreMesh(core_axis_name="c", subcore_axis_name="s",
                                  num_cores=_NCORES, num_subcores=_NTILES)


def _fill_f32(buf, n, value):
    v = jnp.full((_LANE,), value, jnp.float32)

    def body(i, carry):
        buf[pl.ds(i * _LANE, _LANE)] = v
        return carry

    lax.fori_loop(0, n // _LANE, body, 0)


def _zero_rows(buf, nrows, width):
    z = jnp.zeros((_LANE,), jnp.float32)

    def body(i, carry):
        for k in range(width // _LANE):
            buf[i, pl.ds(k * _LANE, _LANE)] = z
        return carry

    lax.fori_loop(0, nrows, body, 0)


@functools.lru_cache(maxsize=None)
def _make_deg_hist(n_pad, nchunks, interpret=False):
    stripe = n_pad // _NTILES

    @functools.partial(
        pl.kernel,
        out_type=jax.ShapeDtypeStruct((_NCORES * n_pad,), jnp.float32),
        mesh=_mesh(),
        scratch_types=[
            pltpu.VMEM((nchunks, 128), jnp.int32),
            pltpu.VMEM((128,), jnp.float32),
            pltpu.VMEM((stripe,), jnp.float32),
            pltpu.VMEM_SHARED((n_pad,), jnp.float32),
        ],
        compiler_params=pltpu.CompilerParams(use_tc_tiling_on_sc=False,
                                             needs_layout_passes=False),
        interpret=interpret,
    )
    def deg_hist(dst_hbm, out_hbm, idx_v, ones_v, zb_v, hist_sh):
        c = lax.axis_index("c")
        s = lax.axis_index("s")
        wid = c * _NTILES + s
        _fill_f32(ones_v, 128, 1.0)
        _fill_f32(zb_v, stripe, 0.0)
        pltpu.sync_copy(dst_hbm.at[pl.ds(wid * nchunks, nchunks)], idx_v)
        pltpu.sync_copy(zb_v, hist_sh.at[pl.ds(s * stripe, stripe)])
        plsc.subcore_barrier()

        def body(j, carry):
            pltpu.sync_copy(ones_v, hist_sh.at[idx_v.at[j]], add=True)
            return carry

        lax.fori_loop(0, nchunks, body, 0)
        plsc.subcore_barrier()
        pltpu.sync_copy(hist_sh.at[pl.ds(s * stripe, stripe)],
                        out_hbm.at[pl.ds(c * n_pad + s * stripe, stripe)])

    return deg_hist


@functools.lru_cache(maxsize=None)
def _make_edge_acc(n_pad, width, nchunks, interpret=False):
    stripe = n_pad // _NTILES
    zrows = 400 if stripe % 400 == 0 else stripe
    nz = stripe // zrows

    @functools.partial(
        pl.kernel,
        out_type=[jax.ShapeDtypeStruct((n_pad, width), jnp.float32),
                  jax.ShapeDtypeStruct((n_pad, width), jnp.float32)],
        mesh=_mesh(),
        scratch_types=[
            pltpu.VMEM((nchunks, 128), jnp.int32),
            pltpu.VMEM((nchunks, 128), jnp.int32),
            pltpu.VMEM((128, width), jnp.float32),
            pltpu.VMEM((zrows, width), jnp.float32),
            pltpu.SemaphoreType.DMA,
            pltpu.VMEM_SHARED((n_pad, width), jnp.float32),
        ],
        compiler_params=pltpu.CompilerParams(use_tc_tiling_on_sc=False,
                                             needs_layout_passes=False),
        interpret=interpret,
    )
    def edge_acc(rows_hbm, src_hbm, dst_hbm, out0_hbm, out1_hbm,
                 sidx_v, didx_v, rowbuf, zb_v, sem, acc_sh):
        c = lax.axis_index("c")
        s = lax.axis_index("s")
        wid = c * _NTILES + s
        _zero_rows(zb_v, zrows, width)
        pltpu.sync_copy(src_hbm.at[pl.ds(wid * nchunks, nchunks)], sidx_v)
        pltpu.sync_copy(dst_hbm.at[pl.ds(wid * nchunks, nchunks)], didx_v)

        def zbody(k, carry):
            pltpu.sync_copy(
                zb_v, acc_sh.at[pl.ds(s * stripe + k * zrows, zrows)])
            return carry

        lax.fori_loop(0, nz, zbody, 0)
        plsc.subcore_barrier()

        def body(j, carry):
            pltpu.async_copy(rows_hbm.at[sidx_v.at[j]], rowbuf, sem).wait()
            pltpu.sync_copy(rowbuf, acc_sh.at[didx_v.at[j]], add=True)
            return carry

        lax.fori_loop(0, nchunks, body, 0)
        plsc.subcore_barrier()

        @pl.when(c == 0)
        def _():
            pltpu.sync_copy(acc_sh.at[pl.ds(s * stripe, stripe)],
                            out0_hbm.at[pl.ds(s * stripe, stripe)])

        @pl.when(c == 1)
        def _():
            pltpu.sync_copy(acc_sh.at[pl.ds(s * stripe, stripe)],
                            out1_hbm.at[pl.ds(s * stripe, stripe)])

    return edge_acc


@functools.lru_cache(maxsize=None)
def _make_remap_compact(n_pad, nchunks, cap_chunks, nbins, dummy_id,
                        interpret=False):
    stripe = n_pad // _NTILES
    cap = cap_chunks * 128
    nw = _NCORES * _NTILES

    @functools.partial(
        pl.kernel,
        out_type=[jax.ShapeDtypeStruct((nw * cap,), jnp.int32),
                  jax.ShapeDtypeStruct((nw * cap,), jnp.int32),
                  jax.ShapeDtypeStruct((nw * 16,), jnp.int32),
                  jax.ShapeDtypeStruct((nw * nbins,), jnp.float32)],
        mesh=_mesh(),
        scratch_types=[
            pltpu.VMEM((nchunks, 128), jnp.int32),
            pltpu.VMEM((nchunks, 128), jnp.int32),
            pltpu.VMEM((4, 128), jnp.int32),
            pltpu.VMEM((4, 128), jnp.int32),
            pltpu.VMEM((cap + 128,), jnp.int32),
            pltpu.VMEM((cap + 128,), jnp.int32),
            pltpu.VMEM((nbins,), jnp.float32),
            pltpu.VMEM((16,), jnp.int32),
            pltpu.SemaphoreType.DMA,
            pltpu.VMEM_SHARED((n_pad,), jnp.int32),
        ],
        compiler_params=pltpu.CompilerParams(use_tc_tiling_on_sc=False,
                                             needs_layout_passes=False),
        interpret=interpret,
    )
    def remap_compact(map_hbm, src_hbm, dst_hbm,
                      ns_hbm, nd_hbm, cnt_hbm, hist_hbm,
                      sidx_v, didx_v, ms_v, md_v, nsb_v, ndb_v, hist_v,
                      cnt_v, sem, map_sh):
        c = lax.axis_index("c")
        s = lax.axis_index("s")
        wid = c * _NTILES + s
        dummy = jnp.full((_LANE,), dummy_id, jnp.int32)
        ones = jnp.ones((_LANE,), jnp.float32)
        _fill_f32(hist_v, nbins, 0.0)
        pltpu.sync_copy(src_hbm.at[pl.ds(wid * nchunks, nchunks)], sidx_v)
        pltpu.sync_copy(dst_hbm.at[pl.ds(wid * nchunks, nchunks)], didx_v)
        pltpu.sync_copy(map_hbm.at[pl.ds(s * stripe, stripe)],
                        map_sh.at[pl.ds(s * stripe, stripe)])
        plsc.subcore_barrier()

        def group(g, pos):
            base = g * 4
            cps = [pltpu.async_copy(map_sh.at[sidx_v.at[base + t]],
                                    ms_v.at[t], sem) for t in range(4)]
            cps += [pltpu.async_copy(map_sh.at[didx_v.at[base + t]],
                                     md_v.at[t], sem) for t in range(4)]
            for cp in cps:
                cp.wait()
            for t in range(4):
                for v in range(8):
                    ms = ms_v[t, pl.ds(v * _LANE, _LANE)]
                    md = md_v[t, pl.ds(v * _LANE, _LANE)]
                    m = (ms >= 0) & (md >= 0)
                    mi = m.astype(jnp.int32)
                    slot = pos + plsc.cumsum(mi) - mi
                    plsc.store_scatter(nsb_v, [slot], ms, mask=m)
                    plsc.store_scatter(ndb_v, [slot], md, mask=m)
                    plsc.addupdate_scatter(hist_v, [md], ones, mask=m)
                    pos = pos + jnp.sum(mi)
            return pos

        pos = lax.fori_loop(0, nchunks // 4, group, jnp.int32(0))
        for v in range(8):
            nsb_v[pl.ds(pos + v * _LANE, _LANE)] = dummy
            ndb_v[pl.ds(pos + v * _LANE, _LANE)] = dummy
        cnt_v[...] = jnp.full((_LANE,), (pos + 127) // 128, jnp.int32)
        pltpu.sync_copy(nsb_v.at[pl.ds(0, cap)], ns_hbm.at[pl.ds(wid * cap, cap)])
        pltpu.sync_copy(ndb_v.at[pl.ds(0, cap)], nd_hbm.at[pl.ds(wid * cap, cap)])
        pltpu.sync_copy(cnt_v, cnt_hbm.at[pl.ds(wid * 16, 16)])
        pltpu.sync_copy(hist_v, hist_hbm.at[pl.ds(wid * nbins, nbins)])

    return remap_compact


@functools.lru_cache(maxsize=None)
def _make_edge_acc_dyn(n_pad, width, cap_chunks, interpret=False):
    stripe = n_pad // _NTILES
    nw = _NCORES * _NTILES

    @functools.partial(
        pl.kernel,
        out_type=[jax.ShapeDtypeStruct((n_pad, width), jnp.float32),
                  jax.ShapeDtypeStruct((n_pad, width), jnp.float32)],
        mesh=_mesh(),
        scratch_types=[
            pltpu.VMEM((cap_chunks, 128), jnp.int32),
            pltpu.VMEM((cap_chunks, 128), jnp.int32),
            pltpu.VMEM((128, width), jnp.float32),
            pltpu.VMEM((stripe, width), jnp.float32),
            pltpu.VMEM((16,), jnp.int32),
            pltpu.SemaphoreType.DMA,
            pltpu.VMEM_SHARED((n_pad, width), jnp.float32),
        ],
        compiler_params=pltpu.CompilerParams(use_tc_tiling_on_sc=False,
                                             needs_layout_passes=False),
        interpret=interpret,
    )
    def edge_acc_dyn(rows_hbm, src_hbm, dst_hbm, cnt_hbm, out0_hbm, out1_hbm,
                     sidx_v, didx_v, rowbuf, zb_v, cnt_v, sem, acc_sh):
        c = lax.axis_index("c")
        s = lax.axis_index("s")
        wid = c * _NTILES + s
        _zero_rows(zb_v, stripe, width)
        pltpu.sync_copy(src_hbm.at[pl.ds(wid * cap_chunks, cap_chunks)], sidx_v)
        pltpu.sync_copy(dst_hbm.at[pl.ds(wid * cap_chunks, cap_chunks)], didx_v)
        pltpu.sync_copy(cnt_hbm.at[pl.ds(wid * 16, 16)], cnt_v)
        pltpu.sync_copy(zb_v, acc_sh.at[pl.ds(s * stripe, stripe)])
        plsc.subcore_barrier()
        ncj = jnp.max(cnt_v[...])

        def body(j, carry):
            pltpu.async_copy(rows_hbm.at[sidx_v.at[j]], rowbuf, sem).wait()
            pltpu.sync_copy(rowbuf, acc_sh.at[didx_v.at[j]], add=True)
            return carry

        lax.fori_loop(0, ncj, body, 0)
        plsc.subcore_barrier()

        @pl.when(c == 0)
        def _():
            pltpu.sync_copy(acc_sh.at[pl.ds(s * stripe, stripe)],
                            out0_hbm.at[pl.ds(s * stripe, stripe)])

        @pl.when(c == 1)
        def _():
            pltpu.sync_copy(acc_sh.at[pl.ds(s * stripe, stripe)],
                            out1_hbm.at[pl.ds(s * stripe, stripe)])

    return edge_acc_dyn


def _leaky(x):
    return jnp.where(x >= 0, x, 0.01 * x)


def _fc_body(flat_ref, w_ref, b_ref, o_ref):
    o_ref[...] = jnp.dot(flat_ref[...], w_ref[...],
                         preferred_element_type=jnp.float32) + b_ref[...]


def _fc(flat, Wfc, bfc):
    out = pl.pallas_call(
        _fc_body,
        out_shape=jax.ShapeDtypeStruct((1, 128), jnp.float32),
    )(flat.reshape(1, -1), Wfc, bfc.reshape(1, -1))
    return out.reshape(-1)


def kernel(pos, edge_index, W1, b1, p1, W2, b2, W3, b3, p2, Wfc, bfc):
    src = edge_index[0]
    dst = edge_index[1]
    N = pos.shape[0]
    E = src.shape[0]

    n_pad = 51200
    ept = 25600
    e_pad = 32 * ept
    nchunks = ept // 128
    dummy = jnp.int32(N)

    src_p = jnp.full((e_pad,), dummy, jnp.int32).at[:E].set(src)
    dst_p = jnp.full((e_pad,), dummy, jnp.int32).at[:E].set(dst)
    src2d = src_p.reshape(e_pad // 128, 128)
    dst2d = dst_p.reshape(e_pad // 128, 128)

    lin1 = pos @ W1
    hist = _make_deg_hist(n_pad, nchunks)(dst2d).reshape(2, n_pad)
    deg1 = 1.0 + (hist[0] + hist[1])[:N]
    dinv1 = lax.rsqrt(deg1)
    lins1 = jnp.zeros((n_pad, 16), jnp.float32).at[:N].set(
        lin1 * dinv1[:, None])
    acc_a, acc_b = _make_edge_acc(n_pad, 16, nchunks)(lins1, src2d, dst2d)
    acc1 = (acc_a + acc_b)[:N]
    x1 = _leaky(dinv1[:, None] * acc1 + lin1 * (dinv1 * dinv1)[:, None] + b1)

    score1 = (x1 @ p1) / jnp.linalg.norm(p1)
    _, perm1 = lax.top_k(score1, 4096)
    gate1 = jnp.tanh(score1[perm1])
    xs = x1[perm1] * gate1[:, None]
    mapping = jnp.full((n_pad,), -1, jnp.int32).at[perm1].set(
        jnp.arange(4096, dtype=jnp.int32))

    cap_chunks = nchunks + 2
    nbins = 8192
    n2pad = 4224
    ns_c, nd_c, cnts, histp = _make_remap_compact(
        n_pad, nchunks, cap_chunks, nbins, 4096)(mapping, src2d, dst2d)
    ns2d = ns_c.reshape(-1, 128)
    nd2d = nd_c.reshape(-1, 128)
    deg2 = 1.0 + jnp.sum(histp.reshape(32, nbins), axis=0)[:4096]
    dinv2 = lax.rsqrt(deg2)

    lin2 = xs @ W2
    lins2 = jnp.zeros((n2pad, 32), jnp.float32).at[:4096].set(
        lin2 * dinv2[:, None])
    acc2a, acc2b = _make_edge_acc_dyn(n2pad, 32, cap_chunks)(
        lins2, ns2d, nd2d, cnts)
    acc2 = (acc2a + acc2b)[:4096]
    x2 = _leaky(dinv2[:, None] * acc2 + lin2 * (dinv2 * dinv2)[:, None] + b2)

    lin3 = x2 @ W3
    lins3 = jnp.zeros((n2pad, 32), jnp.float32).at[:4096].set(
        lin3 * dinv2[:, None])
    acc3a, acc3b = _make_edge_acc_dyn(n2pad, 32, cap_chunks)(
        lins3, ns2d, nd2d, cnts)
    acc3 = (acc3a + acc3b)[:4096]
    x3 = _leaky(dinv2[:, None] * acc3 + lin3 * (dinv2 * dinv2)[:, None] + b3)

    score2 = (x3 @ p2) / jnp.linalg.norm(p2)
    _, perm2 = lax.top_k(score2, 128)
    xf = x3[perm2] * jnp.tanh(score2[perm2])[:, None]
    flat = xf.T.reshape(-1)
    return _fc(flat, Wfc, bfc)
